# baseline (speedup 1.0000x reference)
import functools

import jax
import jax.numpy as jnp
from jax import lax
from jax.experimental import pallas as pl
from jax.experimental.pallas import tpu as pltpu
from jax.experimental.pallas import tpu_sc as plsc

_B, _G, _H = 8, 64, 256
_P = _B * _G


def _sc_body(ptsT_hbm, gridb_hbm, win_hbm, x_v, y_v, z_v, grid_v, win_v, sem):
    c = lax.axis_index("c")
    s = lax.axis_index("s")
    wid = s * 2 + c

    @pl.when(wid < _B)
    def _():
        base = wid * _G
        pltpu.sync_copy(ptsT_hbm.at[pl.ds(base, _G)], x_v)
        pltpu.sync_copy(ptsT_hbm.at[pl.ds(_P + base, _G)], y_v)
        pltpu.sync_copy(ptsT_hbm.at[pl.ds(2 * _P + base, _G)], z_v)
        pltpu.sync_copy(gridb_hbm, grid_v)

        xs = [x_v[pl.ds(g * 16, 16)] for g in range(4)]
        ys = [y_v[pl.ds(g * 16, 16)] for g in range(4)]
        zs = [z_v[pl.ds(g * 16, 16)] for g in range(4)]

        lane = lax.broadcasted_iota(jnp.int32, (16,), 0)
        big = jnp.full((16,), 1e30, jnp.float32)
        zero_i = jnp.zeros((16,), jnp.int32)

        bests = [big] * 4
        bidxs = [zero_i] * 4
        for j in range(_G):
            gx = grid_v[pl.ds(j * 48, 16)]
            gy = grid_v[pl.ds(j * 48 + 16, 16)]
            gz = grid_v[pl.ds(j * 48 + 32, 16)]
            for g in range(4):
                dx = xs[g] - gx
                dy = ys[g] - gy
                dz = zs[g] - gz
                dj = dx * dx + dy * dy + dz * dz
                m = dj < bests[g]
                bests[g] = jnp.where(m, dj, bests[g])
                bidxs[g] = jnp.where(m, zero_i + j, bidxs[g])

        neg1 = zero_i - 1
        slot_lanes = [lane + sg * 16 for sg in range(4)]
        w = [neg1] * 4
        for i in range(_G):
            g, l = divmod(i, 16)
            idx_i = bidxs[g][l]
            for sg in range(4):
                m = slot_lanes[sg] == idx_i
                w[sg] = jnp.where(m, zero_i + (base + i), w[sg])
        for sg in range(4):
            win_v[pl.ds(sg * 16, 16)] = w[sg]
        pltpu.sync_copy(win_v, win_hbm.at[pl.ds(base, _G)])


def _sc_retrieval(ptsT, gridb_flat):
    mesh = plsc.VectorSubcoreMesh(core_axis_name="c", subcore_axis_name="s")
    fn = functools.partial(
        pl.kernel, mesh=mesh,
        out_type=jax.ShapeDtypeStruct((_P,), jnp.int32),
        scratch_types=[
            pltpu.VMEM((_G,), jnp.float32),
            pltpu.VMEM((_G,), jnp.float32),
            pltpu.VMEM((_G,), jnp.float32),
            pltpu.VMEM((3 * 16 * _G,), jnp.float32),
            pltpu.VMEM((_G,), jnp.int32),
            pltpu.SemaphoreType.DMA,
        ],
    )(_sc_body)
    return fn(ptsT, gridb_flat)


def _dot(a, b, precision=jax.lax.Precision.DEFAULT):
    return jax.lax.dot_general(
        a, b, (((1,), (0,)), ((), ())),
        precision=precision,
        preferred_element_type=jnp.float32)


def _relu(x):
    return jnp.maximum(x, 0.0)


def _tc_encoder_kernel(pts_ref, fts_ref,
                       pe_W1, pe_b1, pe_W2, pe_b2, pe_W3, pe_b3,
                       fe_W1, fe_b1, fe_W2, fe_b2,
                       combined_ref):
    pts = pts_ref[...]
    pf = _relu(_dot(pts, pe_W1[...]) + pe_b1[...])
    pf = _relu(_dot(pf, pe_W2[...]) + pe_b2[...])
    pf = _dot(pf, pe_W3[...]) + pe_b3[...]
    fe = _relu(_dot(fts_ref[...], fe_W1[...]) + fe_b1[...])
    fe = _dot(fe, fe_W2[...]) + fe_b2[...]
    combined_ref[...] = jnp.concatenate([pf, fe], axis=1)


def _tc_kernel(combined_ref, win_ref,
               rn_W1, rn_b1, rn_W2, rn_b2,
               cl_W1, cl_b1, cl_W2, cl_b2,
               po_W1, po_b1, po_W2, po_b2,
               sz_W1, sz_b1, sz_W2, sz_b2,
               probs_ref, pose_ref, size_ref, proc_ref, gf_ref):
    combined = combined_ref[...]
    p2 = jax.lax.broadcasted_iota(jnp.int32, (_P, _P), 1)
    onehot = (p2 == win_ref[...]).astype(jnp.float32)
    gff = _dot(onehot, combined, jax.lax.Precision.HIGHEST)
    gf_ref[...] = gff

    h = _relu(_dot(gff, rn_W1[...]) + rn_b1[...])
    procf = _dot(h, rn_W2[...]) + rn_b2[...]
    proc_ref[...] = procf

    bq = jax.lax.broadcasted_iota(jnp.int32, (_B, _P), 0)
    bp = jax.lax.broadcasted_iota(jnp.int32, (_B, _P), 1) >> 6
    avg = jnp.where(bq == bp, 1.0 / _G, 0.0).astype(jnp.float32)
    agg = _dot(avg, procf, jax.lax.Precision.HIGHEST)

    c1 = _relu(_dot(agg, cl_W1[...]) + cl_b1[...])
    logits = _dot(c1, cl_W2[...]) + cl_b2[...]
    m = jnp.max(logits, axis=1, keepdims=True)
    e = jnp.exp(logits - m)
    probs_ref[...] = e / jnp.sum(e, axis=1, keepdims=True)
    p1 = _relu(_dot(agg, po_W1[...]) + po_b1[...])
    pose_ref[...] = _dot(p1, po_W2[...]) + po_b2[...]
    s1 = _relu(_dot(agg, sz_W1[...]) + sz_b1[...])
    size_ref[...] = jax.nn.sigmoid(_dot(s1, sz_W2[...]) + sz_b2[...])


def kernel(point_cloud, features, grid_points,
           pe_W1, pe_b1, pe_W2, pe_b2, pe_W3, pe_b3,
           fe_W1, fe_b1, fe_W2, fe_b2,
           rn_W1, rn_b1, rn_W2, rn_b2,
           cl_W1, cl_b1, cl_W2, cl_b2,
           po_W1, po_b1, po_W2, po_b2,
           sz_W1, sz_b1, sz_W2, sz_b2):
    C = cl_W2.shape[1]
    pts = point_cloud[:, :_G, :].reshape(_P, 3)
    fts = features[:, :_G, :].reshape(_P, 64)
    ptsT = pts.T
    gridb = jnp.broadcast_to(grid_points[:, :, None], (_G, 3, 16)).reshape(-1)

    win = _sc_retrieval(ptsT.reshape(-1), gridb)
    win_col = win.reshape(_P, 1)

    biases = [pe_b1, pe_b2, pe_b3, fe_b1, fe_b2, rn_b1, rn_b2,
              cl_b1, cl_b2, po_b1, po_b2, sz_b1, sz_b2]
    (pe_b1, pe_b2, pe_b3, fe_b1, fe_b2, rn_b1, rn_b2,
     cl_b1, cl_b2, po_b1, po_b2, sz_b1, sz_b2) = [
        b.reshape(1, -1) for b in biases]

    combined = pl.pallas_call(
        _tc_encoder_kernel,
        out_shape=jax.ShapeDtypeStruct((_P, 2 * _H), jnp.float32))(
            pts, fts,
            pe_W1, pe_b1, pe_W2, pe_b2, pe_W3, pe_b3,
            fe_W1, fe_b1, fe_W2, fe_b2)

    out_shape = (
        jax.ShapeDtypeStruct((_B, C), jnp.float32),
        jax.ShapeDtypeStruct((_B, 7), jnp.float32),
        jax.ShapeDtypeStruct((_B, 3), jnp.float32),
        jax.ShapeDtypeStruct((_P, _H), jnp.float32),
        jax.ShapeDtypeStruct((_P, 2 * _H), jnp.float32),
    )
    probs, pose, size, procf, gff = pl.pallas_call(
        _tc_kernel, out_shape=out_shape)(
            combined, win_col,
            rn_W1, rn_b1, rn_W2, rn_b2,
            cl_W1, cl_b1, cl_W2, cl_b2,
            po_W1, po_b1, po_W2, po_b2,
            sz_W1, sz_b1, sz_W2, sz_b2)

    proc = procf.reshape(_B, _G, _H).transpose(0, 2, 1)
    gf = gff.reshape(_B, _G, 2 * _H).transpose(0, 2, 1)
    return (probs, pose, size, proc, gf)


# speedup vs baseline: 2.1067x; 2.1067x over previous
import jax
import jax.numpy as jnp
from jax.experimental import pallas as pl

_B, _G, _H = 8, 64, 256
_P = _B * _G


def _dot(a, b, precision=jax.lax.Precision.DEFAULT):
    return jax.lax.dot_general(
        a, b, (((1,), (0,)), ((), ())),
        precision=precision,
        preferred_element_type=jnp.float32)


def _relu(x):
    return jnp.maximum(x, 0.0)


def _fused_kernel(pts_ref, fts_ref, grid_ref,
                  pe_W1, pe_b1, pe_W2, pe_b2, pe_W3, pe_b3,
                  fe_W1, fe_b1, fe_W2, fe_b2,
                  rn_W1, rn_b1, rn_W2, rn_b2,
                  cl_W1, cl_b1, cl_W2, cl_b2,
                  po_W1, po_b1, po_W2, po_b2,
                  sz_W1, sz_b1, sz_W2, sz_b2,
                  probs_ref, pose_ref, size_ref, proc_ref, gf_ref):
    pts = pts_ref[...]
    pf = _relu(_dot(pts, pe_W1[...]) + pe_b1[...])
    pf = _relu(_dot(pf, pe_W2[...]) + pe_b2[...])
    pf = _dot(pf, pe_W3[...]) + pe_b3[...]
    fe = _relu(_dot(fts_ref[...], fe_W1[...]) + fe_b1[...])
    fe = _dot(fe, fe_W2[...]) + fe_b2[...]
    combined = jnp.concatenate([pf, fe], axis=1)

    ptsT = pts.T
    gx = grid_ref[:, 0:1]
    gy = grid_ref[:, 1:2]
    gz = grid_ref[:, 2:3]
    dx = gx - ptsT[0:1, :]
    dy = gy - ptsT[1:2, :]
    dz = gz - ptsT[2:3, :]
    d = jnp.sqrt(dx * dx + dy * dy + dz * dz)
    dmin = jnp.min(d, axis=0, keepdims=True)
    j_iota = jax.lax.broadcasted_iota(jnp.int32, (_G, _P), 0)
    idx = jnp.min(jnp.where(d == dmin, j_iota, _G), axis=0, keepdims=True)

    q2 = jax.lax.broadcasted_iota(jnp.int32, (_P, _P), 0)
    p2 = jax.lax.broadcasted_iota(jnp.int32, (_P, _P), 1)
    cond = ((q2 >> 6) == (p2 >> 6)) & (idx == (q2 & (_G - 1)))
    win = jnp.max(jnp.where(cond, p2, -1), axis=1, keepdims=True)
    onehot = (p2 == win).astype(jnp.float32)
    gff = _dot(onehot, combined, jax.lax.Precision.HIGHEST)
    gf_ref[...] = gff

    h = _relu(_dot(gff, rn_W1[...]) + rn_b1[...])
    procf = _dot(h, rn_W2[...]) + rn_b2[...]
    proc_ref[...] = procf

    bq = jax.lax.broadcasted_iota(jnp.int32, (_B, _P), 0)
    bp = jax.lax.broadcasted_iota(jnp.int32, (_B, _P), 1) >> 6
    avg = jnp.where(bq == bp, 1.0 / _G, 0.0).astype(jnp.float32)
    agg = _dot(avg, procf, jax.lax.Precision.HIGHEST)

    c1 = _relu(_dot(agg, cl_W1[...]) + cl_b1[...])
    logits = _dot(c1, cl_W2[...]) + cl_b2[...]
    m = jnp.max(logits, axis=1, keepdims=True)
    e = jnp.exp(logits - m)
    probs_ref[...] = e / jnp.sum(e, axis=1, keepdims=True)
    p1 = _relu(_dot(agg, po_W1[...]) + po_b1[...])
    pose_ref[...] = _dot(p1, po_W2[...]) + po_b2[...]
    s1 = _relu(_dot(agg, sz_W1[...]) + sz_b1[...])
    size_ref[...] = jax.nn.sigmoid(_dot(s1, sz_W2[...]) + sz_b2[...])


def kernel(point_cloud, features, grid_points,
           pe_W1, pe_b1, pe_W2, pe_b2, pe_W3, pe_b3,
           fe_W1, fe_b1, fe_W2, fe_b2,
           rn_W1, rn_b1, rn_W2, rn_b2,
           cl_W1, cl_b1, cl_W2, cl_b2,
           po_W1, po_b1, po_W2, po_b2,
           sz_W1, sz_b1, sz_W2, sz_b2):
    C = cl_W2.shape[1]
    pts = point_cloud[:, :_G, :].reshape(_P, 3)
    fts = features[:, :_G, :].reshape(_P, 64)

    biases = [pe_b1, pe_b2, pe_b3, fe_b1, fe_b2, rn_b1, rn_b2,
              cl_b1, cl_b2, po_b1, po_b2, sz_b1, sz_b2]
    (pe_b1, pe_b2, pe_b3, fe_b1, fe_b2, rn_b1, rn_b2,
     cl_b1, cl_b2, po_b1, po_b2, sz_b1, sz_b2) = [
        b.reshape(1, -1) for b in biases]

    out_shape = (
        jax.ShapeDtypeStruct((_B, C), jnp.float32),
        jax.ShapeDtypeStruct((_B, 7), jnp.float32),
        jax.ShapeDtypeStruct((_B, 3), jnp.float32),
        jax.ShapeDtypeStruct((_P, _H), jnp.float32),
        jax.ShapeDtypeStruct((_P, 2 * _H), jnp.float32),
    )
    probs, pose, size, procf, gff = pl.pallas_call(
        _fused_kernel, out_shape=out_shape)(
            pts, fts, grid_points,
            pe_W1, pe_b1, pe_W2, pe_b2, pe_W3, pe_b3,
            fe_W1, fe_b1, fe_W2, fe_b2,
            rn_W1, rn_b1, rn_W2, rn_b2,
            cl_W1, cl_b1, cl_W2, cl_b2,
            po_W1, po_b1, po_W2, po_b2,
            sz_W1, sz_b1, sz_W2, sz_b2)

    proc = procf.reshape(_B, _G, _H).transpose(0, 2, 1)
    gf = gff.reshape(_B, _G, 2 * _H).transpose(0, 2, 1)
    return (probs, pose, size, proc, gf)
